# baseline (speedup 1.0000x reference)
import functools

import jax
import jax.numpy as jnp
from jax import lax
from jax.experimental import pallas as pl
from jax.experimental.pallas import tpu as pltpu
from jax.experimental.pallas import tpu_sc as plsc

N_JOB = 10000
N_CO = 10000
E = 320000
D = 128
NEG_SLOPE = 0.2

NC = 2
NS = 16
NW = NC * NS
EPT = E // NW
K = 80
NCHUNK = EPT // K
RPT = N_JOB // NS
KB = K
NCB = NCHUNK
NMETA = 4
NSPLIT = 5
DSTRIPE = 624
DREM = N_JOB - NS * DSTRIPE
L = 16


def _attn_vecs_body(xco, wco, bco, xjo, wjo, bjo, ls, ld, as_o, ad_o):
    vs = jnp.dot(wco[...], ls[...], preferred_element_type=jnp.float32)
    cs = jnp.dot(bco[...], ls[...], preferred_element_type=jnp.float32)
    as_o[...] = jnp.dot(xco[...], vs,
                        preferred_element_type=jnp.float32)[:, 0] + cs[0, 0]
    vd = jnp.dot(wjo[...], ld[...], preferred_element_type=jnp.float32)
    cd = jnp.dot(bjo[...], ld[...], preferred_element_type=jnp.float32)
    ad_o[...] = jnp.dot(xjo[...], vd,
                        preferred_element_type=jnp.float32)[:, 0] + cd[0, 0]


def _attn_vecs(x_co, w_co, b_co, x_jo, w_jo, b_jo, ls, ld):
    return pl.pallas_call(
        _attn_vecs_body,
        out_shape=[
            jax.ShapeDtypeStruct((N_CO,), jnp.float32),
            jax.ShapeDtypeStruct((N_JOB,), jnp.float32),
        ],
    )(x_co, w_co, b_co, x_jo, w_jo, b_jo, ls, ld)


def _proj_body(xco, wco, bco, xc_o):
    xc = jnp.dot(xco[...], wco[...], preferred_element_type=jnp.float32)
    xc_o[...] = xc + bco[...]


def _proj(x_co, w_co, b_co):
    return pl.pallas_call(
        _proj_body,
        out_shape=jax.ShapeDtypeStruct((N_CO, D), jnp.float32),
    )(x_co, w_co, b_co)


def _sc_logits_body(asrc_hbm, adst_hbm, ei_hbm,
                    e_out, s_out,
                    src_f, dst_f, e_f, asrc_v, adst_v, s_v):
    cid = lax.axis_index("c")
    sid = lax.axis_index("s")
    wid = cid * NS + sid
    off = wid * EPT

    pltpu.sync_copy(ei_hbm.at[pl.ds(off, EPT)], src_f)
    pltpu.sync_copy(ei_hbm.at[pl.ds(E + off, EPT)], dst_f)
    pltpu.sync_copy(asrc_hbm, asrc_v)
    pltpu.sync_copy(adst_hbm, adst_v)

    z16 = jnp.zeros((L,), jnp.float32)

    def _zs(i, _):
        s_v[pl.ds(i * L, L)] = z16
        return 0
    lax.fori_loop(0, N_JOB // L, _zs, 0)

    def _sub(j, _):
        isrc = src_f[pl.ds(j * L, L)]
        idst = dst_f[pl.ds(j * L, L)]
        av = plsc.load_gather(asrc_v, [isrc])
        bv = plsc.load_gather(adst_v, [idst])
        al = av + bv
        al = jnp.where(al >= 0, al, NEG_SLOPE * al)
        ev = jnp.exp(al)
        e_f[pl.ds(j * L, L)] = ev
        plsc.addupdate_scatter(s_v, [idst], ev)
        return 0
    lax.fori_loop(0, EPT // L, _sub, 0, unroll=5)

    pltpu.sync_copy(e_f, e_out.at[pl.ds(off, EPT)])
    pltpu.sync_copy(s_v, s_out.at[wid])


def _sc_logits(a_src, a_dst, ei):
    mesh = plsc.VectorSubcoreMesh(core_axis_name="c", subcore_axis_name="s")
    fn = pl.kernel(
        _sc_logits_body,
        out_type=[
            jax.ShapeDtypeStruct((E,), jnp.float32),
            jax.ShapeDtypeStruct((NW, N_JOB), jnp.float32),
        ],
        mesh=mesh,
        scratch_types=[
            pltpu.VMEM((EPT,), jnp.int32),
            pltpu.VMEM((EPT,), jnp.int32),
            pltpu.VMEM((EPT,), jnp.float32),
            pltpu.VMEM((N_CO,), jnp.float32),
            pltpu.VMEM((N_JOB,), jnp.float32),
            pltpu.VMEM((N_JOB,), jnp.float32),
        ],
        compiler_params=pltpu.CompilerParams(needs_layout_passes=False),
    )
    return fn(a_src, a_dst, ei)


def _sc_scatter_body(xc_hbm, ei_hbm, e_hbm,
                     u_out,
                     sidx, didx, ebuf, rows_v, shared_u,
                     semi, semg, sems):
    cid = lax.axis_index("c")
    sid = lax.axis_index("s")
    wid = cid * NS + sid
    eoff = wid * EPT

    z16 = jnp.zeros((L,), jnp.float32)

    def _zrow(k, _):
        for r in range(D // L):
            rows_v[0, k, pl.ds(r * L, L)] = z16
        return 0
    lax.fori_loop(0, KB, _zrow, 0)

    base = sid * RPT

    def _zchunk(c, _):
        pltpu.sync_copy(rows_v.at[0], shared_u.at[pl.ds(base + c * KB, KB)])
        return 0
    lax.fori_loop(0, RPT // KB, _zchunk, 0)
    rem = RPT - (RPT // KB) * KB
    if rem:
        pltpu.sync_copy(rows_v.at[0, pl.ds(0, rem)],
                        shared_u.at[pl.ds(base + (RPT // KB) * KB, rem)])

    plsc.subcore_barrier()

    HB = KB // NSPLIT

    def _load_meta(c, slot):
        pltpu.async_copy(ei_hbm.at[pl.ds(eoff + c * KB, KB)],
                         sidx.at[slot], semi)
        for q in range(NSPLIT):
            pltpu.async_copy(
                ei_hbm.at[pl.ds(E + eoff + c * KB + q * HB, HB)],
                didx.at[NSPLIT * slot + q], semi)
        pltpu.async_copy(e_hbm.at[pl.ds(eoff + c * KB, KB)],
                         ebuf.at[slot], semi)

    def _wait_meta(c, slot):
        pltpu.make_async_copy(ei_hbm.at[pl.ds(eoff + c * KB, KB)],
                              sidx.at[slot], semi).wait()
        for q in range(NSPLIT):
            pltpu.make_async_copy(
                ei_hbm.at[pl.ds(E + eoff + c * KB + q * HB, HB)],
                didx.at[NSPLIT * slot + q], semi).wait()
        pltpu.make_async_copy(e_hbm.at[pl.ds(eoff + c * KB, KB)],
                              ebuf.at[slot], semi).wait()

    _load_meta(0, 0)
    _wait_meta(0, 0)
    pltpu.async_copy(xc_hbm.at[sidx.at[0]], rows_v.at[0], semg)
    _load_meta(1, 1)

    def _chunk_b(c, _):
        b = lax.rem(c, 2)
        nb = 1 - b
        m3 = lax.rem(c, NMETA)
        n3 = lax.rem(c + 1, NMETA)
        p3 = lax.rem(c + 2, NMETA)

        pltpu.make_async_copy(xc_hbm.at[sidx.at[m3]], rows_v.at[b],
                              semg).wait()

        @pl.when(c >= 1)
        def _():
            for q in range(NSPLIT):
                pltpu.make_async_copy(
                    rows_v.at[nb, pl.ds(q * HB, HB)],
                    shared_u.at[didx.at[NSPLIT * n3 + q]], sems).wait()

        @pl.when(c + 1 < NCB)
        def _():
            _wait_meta(c + 1, n3)
            pltpu.async_copy(xc_hbm.at[sidx.at[n3]], rows_v.at[nb], semg)

        @pl.when(c + 2 < NCB)
        def _():
            _load_meta(c + 2, p3)

        i0 = jnp.full((L,), m3, dtype=jnp.int32)

        def _row(k, _):
            ik = jnp.full((L,), k, dtype=jnp.int32)
            es = plsc.load_gather(ebuf, [i0, ik])
            for r in range(D // L):
                sl = pl.ds(r * L, L)
                rows_v[b, k, sl] = rows_v[b, k, sl] * es
            return 0
        for q in range(NSPLIT):
            lax.fori_loop(q * HB, (q + 1) * HB, _row, 0, unroll=5)
            pltpu.async_copy(rows_v.at[b, pl.ds(q * HB, HB)],
                             shared_u.at[didx.at[NSPLIT * m3 + q]], sems,
                             add=True)
        return 0
    lax.fori_loop(0, NCB, _chunk_b, 0)

    bl = lax.rem(NCB - 1, 2)
    ml = lax.rem(NCB - 1, NMETA)
    for q in range(NSPLIT):
        pltpu.make_async_copy(rows_v.at[bl, pl.ds(q * HB, HB)],
                              shared_u.at[didx.at[NSPLIT * ml + q]],
                              sems).wait()

    plsc.subcore_barrier()

    dbase = sid * DSTRIPE
    pltpu.sync_copy(shared_u.at[pl.ds(dbase, DSTRIPE)],
                    u_out.at[cid, pl.ds(dbase, DSTRIPE)])

    @pl.when(sid == 0)
    def _drain_tail():
        pltpu.sync_copy(shared_u.at[pl.ds(NS * DSTRIPE, DREM)],
                        u_out.at[cid, pl.ds(NS * DSTRIPE, DREM)])


def _sc_scatter(xc, ei, e2):
    mesh = plsc.VectorSubcoreMesh(core_axis_name="c", subcore_axis_name="s")
    fn = pl.kernel(
        _sc_scatter_body,
        out_type=jax.ShapeDtypeStruct((NC, N_JOB, D), jnp.float32),
        mesh=mesh,
        scratch_types=[
            pltpu.VMEM((NMETA, KB), jnp.int32),
            pltpu.VMEM((NSPLIT * NMETA, KB // NSPLIT), jnp.int32),
            pltpu.VMEM((NMETA, KB), jnp.float32),
            pltpu.VMEM((2, KB, D), jnp.float32),
            pltpu.VMEM_SHARED((N_JOB, D), jnp.float32),
            pltpu.SemaphoreType.DMA,
            pltpu.SemaphoreType.DMA,
            pltpu.SemaphoreType.DMA,
        ],
        compiler_params=pltpu.CompilerParams(needs_layout_passes=False),
    )
    return fn(xc, ei, e2)


def _final_body(u_ref, s_ref, w_ref, b_ref, o_ref):
    u = u_ref[0] + u_ref[1]
    r = jnp.maximum(u, 0.0)
    y = jnp.dot(r, w_ref[...], preferred_element_type=jnp.float32)
    s = jnp.sum(s_ref[...], axis=1, keepdims=True)
    o_ref[...] = (y / (s + 1e-16) + b_ref[0, 0])[:, 0]


def _final(u_parts, s_parts_t, mlp_w, mlp_b):
    return pl.pallas_call(
        _final_body,
        out_shape=jax.ShapeDtypeStruct((N_JOB,), jnp.float32),
    )(u_parts, s_parts_t, mlp_w, mlp_b)


@jax.jit
def kernel(x_job, x_company, edge_index_job_to_company, edge_index_company_to_job,
           proj_job_w, proj_job_b, proj_co_w, proj_co_b,
           lin_src_j2c, lin_dst_j2c, lin_src_c2j, lin_dst_c2j,
           k_lin_w, k_lin_b, q, mlp_w, mlp_b):
    del edge_index_job_to_company, lin_src_j2c, lin_dst_j2c, k_lin_w, k_lin_b, q

    ei = edge_index_company_to_job.reshape(2 * E)

    a_src, a_dst = _attn_vecs(
        x_company, proj_co_w, proj_co_b.reshape(1, D),
        x_job, proj_job_w, proj_job_b.reshape(1, D),
        lin_src_c2j.reshape(D, 1), lin_dst_c2j.reshape(D, 1))
    xc = _proj(x_company, proj_co_w, proj_co_b.reshape(1, D))

    e2, s_parts = _sc_logits(a_src, a_dst, ei)
    u_parts = _sc_scatter(xc, ei, e2)

    return _final(u_parts, s_parts.T, mlp_w, mlp_b.reshape(1, 1))


# speedup vs baseline: 1.0420x; 1.0420x over previous
import functools

import jax
import jax.numpy as jnp
from jax import lax
from jax.experimental import pallas as pl
from jax.experimental.pallas import tpu as pltpu
from jax.experimental.pallas import tpu_sc as plsc

N_JOB = 10000
N_CO = 10000
E = 320000
D = 128
NEG_SLOPE = 0.2

NC = 2
NS = 16
NW = NC * NS
EPT = E // NW
K = 80
NCHUNK = EPT // K
RPT = N_JOB // NS
KB = K
NCB = NCHUNK
NMETA = 4
NSPLIT = 2
DSTRIPE = 624
DREM = N_JOB - NS * DSTRIPE
L = 16


def _proj_body(xco, wco, bco, xjo, wjo, bjo, ls, ld, xc_o, as_o, ad_o):
    xc = jnp.dot(xco[...], wco[...], preferred_element_type=jnp.float32)
    xc = xc + bco[...]
    xc_o[...] = xc
    as_o[...] = jnp.dot(xc, ls[...], preferred_element_type=jnp.float32)[:, 0]
    xj = jnp.dot(xjo[...], wjo[...], preferred_element_type=jnp.float32)
    xj = xj + bjo[...]
    ad_o[...] = jnp.dot(xj, ld[...], preferred_element_type=jnp.float32)[:, 0]


def _proj(x_co, w_co, b_co, x_jo, w_jo, b_jo, ls, ld):
    return pl.pallas_call(
        _proj_body,
        out_shape=[
            jax.ShapeDtypeStruct((N_CO, D), jnp.float32),
            jax.ShapeDtypeStruct((N_CO,), jnp.float32),
            jax.ShapeDtypeStruct((N_JOB,), jnp.float32),
        ],
    )(x_co, w_co, b_co, x_jo, w_jo, b_jo, ls, ld)


def _sc_logits_body(asrc_hbm, adst_hbm, ei_hbm,
                    e_out, s_out,
                    src_f, dst_f, e_f, asrc_v, adst_v, s_v):
    cid = lax.axis_index("c")
    sid = lax.axis_index("s")
    wid = cid * NS + sid
    off = wid * EPT

    pltpu.sync_copy(ei_hbm.at[pl.ds(off, EPT)], src_f)
    pltpu.sync_copy(ei_hbm.at[pl.ds(E + off, EPT)], dst_f)
    pltpu.sync_copy(asrc_hbm, asrc_v)
    pltpu.sync_copy(adst_hbm, adst_v)

    z16 = jnp.zeros((L,), jnp.float32)

    def _zs(i, _):
        s_v[pl.ds(i * L, L)] = z16
        return 0
    lax.fori_loop(0, N_JOB // L, _zs, 0)

    def _sub(j, _):
        isrc = src_f[pl.ds(j * L, L)]
        idst = dst_f[pl.ds(j * L, L)]
        av = plsc.load_gather(asrc_v, [isrc])
        bv = plsc.load_gather(adst_v, [idst])
        al = av + bv
        al = jnp.where(al >= 0, al, NEG_SLOPE * al)
        ev = jnp.exp(al)
        e_f[pl.ds(j * L, L)] = ev
        plsc.addupdate_scatter(s_v, [idst], ev)
        return 0
    lax.fori_loop(0, EPT // L, _sub, 0, unroll=5)

    pltpu.sync_copy(e_f, e_out.at[pl.ds(off, EPT)])
    pltpu.sync_copy(s_v, s_out.at[wid])


def _sc_logits(a_src, a_dst, ei):
    mesh = plsc.VectorSubcoreMesh(core_axis_name="c", subcore_axis_name="s")
    fn = pl.kernel(
        _sc_logits_body,
        out_type=[
            jax.ShapeDtypeStruct((E,), jnp.float32),
            jax.ShapeDtypeStruct((NW, N_JOB), jnp.float32),
        ],
        mesh=mesh,
        scratch_types=[
            pltpu.VMEM((EPT,), jnp.int32),
            pltpu.VMEM((EPT,), jnp.int32),
            pltpu.VMEM((EPT,), jnp.float32),
            pltpu.VMEM((N_CO,), jnp.float32),
            pltpu.VMEM((N_JOB,), jnp.float32),
            pltpu.VMEM((N_JOB,), jnp.float32),
        ],
        compiler_params=pltpu.CompilerParams(needs_layout_passes=False),
    )
    return fn(a_src, a_dst, ei)


def _sc_scatter_body(xc_hbm, ei_hbm, e_hbm,
                     u_out,
                     sidx, didx, ebuf, rows_v, shared_u,
                     semi, semg, sems):
    cid = lax.axis_index("c")
    sid = lax.axis_index("s")
    wid = cid * NS + sid
    eoff = wid * EPT

    z16 = jnp.zeros((L,), jnp.float32)

    def _zrow(k, _):
        for r in range(D // L):
            rows_v[0, k, pl.ds(r * L, L)] = z16
        return 0
    lax.fori_loop(0, KB, _zrow, 0)

    base = sid * RPT

    def _zchunk(c, _):
        pltpu.sync_copy(rows_v.at[0], shared_u.at[pl.ds(base + c * KB, KB)])
        return 0
    lax.fori_loop(0, RPT // KB, _zchunk, 0)
    rem = RPT - (RPT // KB) * KB
    if rem:
        pltpu.sync_copy(rows_v.at[0, pl.ds(0, rem)],
                        shared_u.at[pl.ds(base + (RPT // KB) * KB, rem)])

    plsc.subcore_barrier()

    HB = KB // NSPLIT

    def _load_meta(c, slot):
        pltpu.async_copy(ei_hbm.at[pl.ds(eoff + c * KB, KB)],
                         sidx.at[slot], semi)
        for q in range(NSPLIT):
            pltpu.async_copy(
                ei_hbm.at[pl.ds(E + eoff + c * KB + q * HB, HB)],
                didx.at[NSPLIT * slot + q], semi)
        pltpu.async_copy(e_hbm.at[pl.ds(eoff + c * KB, KB)],
                         ebuf.at[slot], semi)

    def _wait_meta(c, slot):
        pltpu.make_async_copy(ei_hbm.at[pl.ds(eoff + c * KB, KB)],
                              sidx.at[slot], semi).wait()
        for q in range(NSPLIT):
            pltpu.make_async_copy(
                ei_hbm.at[pl.ds(E + eoff + c * KB + q * HB, HB)],
                didx.at[NSPLIT * slot + q], semi).wait()
        pltpu.make_async_copy(e_hbm.at[pl.ds(eoff + c * KB, KB)],
                              ebuf.at[slot], semi).wait()

    _load_meta(0, 0)
    _wait_meta(0, 0)
    pltpu.async_copy(xc_hbm.at[sidx.at[0]], rows_v.at[0], semg)
    _load_meta(1, 1)

    def _chunk_b(c, _):
        b = lax.rem(c, 2)
        nb = 1 - b
        m3 = lax.rem(c, NMETA)
        n3 = lax.rem(c + 1, NMETA)
        p3 = lax.rem(c + 2, NMETA)

        pltpu.make_async_copy(xc_hbm.at[sidx.at[m3]], rows_v.at[b],
                              semg).wait()

        @pl.when(c >= 1)
        def _():
            for q in range(NSPLIT):
                pltpu.make_async_copy(
                    rows_v.at[nb, pl.ds(q * HB, HB)],
                    shared_u.at[didx.at[NSPLIT * n3 + q]], sems).wait()

        @pl.when(c + 1 < NCB)
        def _():
            _wait_meta(c + 1, n3)
            pltpu.async_copy(xc_hbm.at[sidx.at[n3]], rows_v.at[nb], semg)

        @pl.when(c + 2 < NCB)
        def _():
            _load_meta(c + 2, p3)

        i0 = jnp.full((L,), m3, dtype=jnp.int32)

        def _row(k, _):
            ik = jnp.full((L,), k, dtype=jnp.int32)
            es = plsc.load_gather(ebuf, [i0, ik])
            for r in range(D // L):
                sl = pl.ds(r * L, L)
                rows_v[b, k, sl] = rows_v[b, k, sl] * es
            return 0
        for q in range(NSPLIT):
            lax.fori_loop(q * HB, (q + 1) * HB, _row, 0, unroll=5)
            pltpu.async_copy(rows_v.at[b, pl.ds(q * HB, HB)],
                             shared_u.at[didx.at[NSPLIT * m3 + q]], sems,
                             add=True)
        return 0
    lax.fori_loop(0, NCB, _chunk_b, 0)

    bl = lax.rem(NCB - 1, 2)
    ml = lax.rem(NCB - 1, NMETA)
    for q in range(NSPLIT):
        pltpu.make_async_copy(rows_v.at[bl, pl.ds(q * HB, HB)],
                              shared_u.at[didx.at[NSPLIT * ml + q]],
                              sems).wait()

    plsc.subcore_barrier()

    dbase = sid * DSTRIPE
    pltpu.sync_copy(shared_u.at[pl.ds(dbase, DSTRIPE)],
                    u_out.at[cid, pl.ds(dbase, DSTRIPE)])

    @pl.when(sid == 0)
    def _drain_tail():
        pltpu.sync_copy(shared_u.at[pl.ds(NS * DSTRIPE, DREM)],
                        u_out.at[cid, pl.ds(NS * DSTRIPE, DREM)])


def _sc_scatter(xc, ei, e2):
    mesh = plsc.VectorSubcoreMesh(core_axis_name="c", subcore_axis_name="s")
    fn = pl.kernel(
        _sc_scatter_body,
        out_type=jax.ShapeDtypeStruct((NC, N_JOB, D), jnp.float32),
        mesh=mesh,
        scratch_types=[
            pltpu.VMEM((NMETA, KB), jnp.int32),
            pltpu.VMEM((NSPLIT * NMETA, KB // NSPLIT), jnp.int32),
            pltpu.VMEM((NMETA, KB), jnp.float32),
            pltpu.VMEM((2, KB, D), jnp.float32),
            pltpu.VMEM_SHARED((N_JOB, D), jnp.float32),
            pltpu.SemaphoreType.DMA,
            pltpu.SemaphoreType.DMA,
            pltpu.SemaphoreType.DMA,
        ],
        compiler_params=pltpu.CompilerParams(needs_layout_passes=False),
    )
    return fn(xc, ei, e2)


def _final_body(u_ref, s_ref, w_ref, b_ref, o_ref):
    u = u_ref[0] + u_ref[1]
    r = jnp.maximum(u, 0.0)
    y = jnp.dot(r, w_ref[...], preferred_element_type=jnp.float32)
    s = jnp.sum(s_ref[...], axis=1, keepdims=True)
    o_ref[...] = (y / (s + 1e-16) + b_ref[0, 0])[:, 0]


def _final(u_parts, s_parts_t, mlp_w, mlp_b):
    return pl.pallas_call(
        _final_body,
        out_shape=jax.ShapeDtypeStruct((N_JOB,), jnp.float32),
    )(u_parts, s_parts_t, mlp_w, mlp_b)


@jax.jit
def kernel(x_job, x_company, edge_index_job_to_company, edge_index_company_to_job,
           proj_job_w, proj_job_b, proj_co_w, proj_co_b,
           lin_src_j2c, lin_dst_j2c, lin_src_c2j, lin_dst_c2j,
           k_lin_w, k_lin_b, q, mlp_w, mlp_b):
    del edge_index_job_to_company, lin_src_j2c, lin_dst_j2c, k_lin_w, k_lin_b, q

    ei = edge_index_company_to_job.reshape(2 * E)

    xc, a_src, a_dst = _proj(
        x_company, proj_co_w, proj_co_b.reshape(1, D),
        x_job, proj_job_w, proj_job_b.reshape(1, D),
        lin_src_c2j.reshape(D, 1), lin_dst_c2j.reshape(D, 1))

    e2, s_parts = _sc_logits(a_src, a_dst, ei)
    u_parts = _sc_scatter(xc, ei, e2)

    return _final(u_parts, s_parts.T, mlp_w, mlp_b.reshape(1, 1))


# speedup vs baseline: 1.0456x; 1.0035x over previous
import jax
import jax.numpy as jnp
from jax import lax
from jax.experimental import pallas as pl
from jax.experimental.pallas import tpu as pltpu
from jax.experimental.pallas import tpu_sc as plsc

N_JOB = 10000
N_CO = 10000
E = 320000
D = 128
NEG_SLOPE = 0.2

NC = 2
NS = 16
NW = NC * NS
EPT = E // NW
K = 80
NCHUNK = EPT // K
RPT = N_JOB // NS
KB = K
NCB = NCHUNK
NMETA = 4
NSPLIT = 2
DSTRIPE = 624
DREM = N_JOB - NS * DSTRIPE
L = 16


def _proj_body(xco, wco, bco, xjo, wjo, bjo, ls, ld, xc_o, as_o, ad_o):
    xc = jnp.dot(xco[...], wco[...], preferred_element_type=jnp.float32)
    xc = xc + bco[...]
    xc_o[...] = xc
    as_o[...] = jnp.dot(xc, ls[...], preferred_element_type=jnp.float32)[:, 0]
    xj = jnp.dot(xjo[...], wjo[...], preferred_element_type=jnp.float32)
    xj = xj + bjo[...]
    ad_o[...] = jnp.dot(xj, ld[...], preferred_element_type=jnp.float32)[:, 0]


def _proj(x_co, w_co, b_co, x_jo, w_jo, b_jo, ls, ld):
    return pl.pallas_call(
        _proj_body,
        out_shape=[
            jax.ShapeDtypeStruct((N_CO, D), jnp.float32),
            jax.ShapeDtypeStruct((N_CO,), jnp.float32),
            jax.ShapeDtypeStruct((N_JOB,), jnp.float32),
        ],
    )(x_co, w_co, b_co, x_jo, w_jo, b_jo, ls, ld)


def _sc_logits_body(asrc_hbm, adst_hbm, ei_hbm,
                    e_out, s_out,
                    src_f, dst_f, e_f, asrc_v, adst_v, s_v):
    cid = lax.axis_index("c")
    sid = lax.axis_index("s")
    wid = cid * NS + sid
    off = wid * EPT

    pltpu.sync_copy(ei_hbm.at[pl.ds(off, EPT)], src_f)
    pltpu.sync_copy(ei_hbm.at[pl.ds(E + off, EPT)], dst_f)
    pltpu.sync_copy(asrc_hbm, asrc_v)
    pltpu.sync_copy(adst_hbm, adst_v)

    z16 = jnp.zeros((L,), jnp.float32)

    def _zs(i, _):
        s_v[pl.ds(i * L, L)] = z16
        return 0
    lax.fori_loop(0, N_JOB // L, _zs, 0)

    def _sub(j, _):
        isrc = src_f[pl.ds(j * L, L)]
        idst = dst_f[pl.ds(j * L, L)]
        av = plsc.load_gather(asrc_v, [isrc])
        bv = plsc.load_gather(adst_v, [idst])
        al = av + bv
        al = jnp.where(al >= 0, al, NEG_SLOPE * al)
        ev = jnp.exp(al)
        e_f[pl.ds(j * L, L)] = ev
        plsc.addupdate_scatter(s_v, [idst], ev)
        return 0
    lax.fori_loop(0, EPT // L, _sub, 0, unroll=5)

    pltpu.sync_copy(e_f, e_out.at[pl.ds(off, EPT)])
    pltpu.sync_copy(s_v, s_out.at[wid])


def _sc_logits(a_src, a_dst, ei):
    mesh = plsc.VectorSubcoreMesh(core_axis_name="c", subcore_axis_name="s")
    fn = pl.kernel(
        _sc_logits_body,
        out_type=[
            jax.ShapeDtypeStruct((E,), jnp.float32),
            jax.ShapeDtypeStruct((NW, N_JOB), jnp.float32),
        ],
        mesh=mesh,
        scratch_types=[
            pltpu.VMEM((EPT,), jnp.int32),
            pltpu.VMEM((EPT,), jnp.int32),
            pltpu.VMEM((EPT,), jnp.float32),
            pltpu.VMEM((N_CO,), jnp.float32),
            pltpu.VMEM((N_JOB,), jnp.float32),
            pltpu.VMEM((N_JOB,), jnp.float32),
        ],
        compiler_params=pltpu.CompilerParams(needs_layout_passes=False),
    )
    return fn(a_src, a_dst, ei)


def _sc_scatter_body(xc_hbm, ei_hbm, e_hbm,
                     u_out,
                     sidx, didx, ebuf, rows_v, shared_u,
                     semi, semg, sems):
    cid = lax.axis_index("c")
    sid = lax.axis_index("s")
    wid = cid * NS + sid
    eoff = wid * EPT

    z16 = jnp.zeros((L,), jnp.float32)

    def _zrow(k, _):
        for r in range(D // L):
            rows_v[0, k, pl.ds(r * L, L)] = z16
        return 0
    lax.fori_loop(0, KB, _zrow, 0)

    base = sid * RPT

    def _zchunk(c, _):
        pltpu.sync_copy(rows_v.at[0], shared_u.at[pl.ds(base + c * KB, KB)])
        return 0
    lax.fori_loop(0, RPT // KB, _zchunk, 0)
    rem = RPT - (RPT // KB) * KB
    if rem:
        pltpu.sync_copy(rows_v.at[0, pl.ds(0, rem)],
                        shared_u.at[pl.ds(base + (RPT // KB) * KB, rem)])

    plsc.subcore_barrier()

    HB = KB // NSPLIT

    def _load_meta(c, slot):
        pltpu.async_copy(ei_hbm.at[pl.ds(eoff + c * KB, KB)],
                         sidx.at[slot], semi)
        for q in range(NSPLIT):
            pltpu.async_copy(
                ei_hbm.at[pl.ds(E + eoff + c * KB + q * HB, HB)],
                didx.at[NSPLIT * slot + q], semi)
        pltpu.async_copy(e_hbm.at[pl.ds(eoff + c * KB, KB)],
                         ebuf.at[slot], semi)

    def _wait_meta(c, slot):
        pltpu.make_async_copy(ei_hbm.at[pl.ds(eoff + c * KB, KB)],
                              sidx.at[slot], semi).wait()
        for q in range(NSPLIT):
            pltpu.make_async_copy(
                ei_hbm.at[pl.ds(E + eoff + c * KB + q * HB, HB)],
                didx.at[NSPLIT * slot + q], semi).wait()
        pltpu.make_async_copy(e_hbm.at[pl.ds(eoff + c * KB, KB)],
                              ebuf.at[slot], semi).wait()

    _load_meta(0, 0)
    _wait_meta(0, 0)
    pltpu.async_copy(xc_hbm.at[sidx.at[0]], rows_v.at[0], semg)
    _load_meta(1, 1)

    def _chunk_b(c, _):
        b = lax.rem(c, 2)
        nb = 1 - b
        m3 = lax.rem(c, NMETA)
        n3 = lax.rem(c + 1, NMETA)
        p3 = lax.rem(c + 2, NMETA)

        pltpu.make_async_copy(xc_hbm.at[sidx.at[m3]], rows_v.at[b],
                              semg).wait()

        @pl.when(c >= 1)
        def _():
            for q in range(NSPLIT):
                pltpu.make_async_copy(
                    rows_v.at[nb, pl.ds(q * HB, HB)],
                    shared_u.at[didx.at[NSPLIT * n3 + q]], sems).wait()

        @pl.when(c + 1 < NCB)
        def _():
            _wait_meta(c + 1, n3)
            pltpu.async_copy(xc_hbm.at[sidx.at[n3]], rows_v.at[nb], semg)

        @pl.when(c + 2 < NCB)
        def _():
            _load_meta(c + 2, p3)

        i0 = jnp.full((L,), m3, dtype=jnp.int32)

        def _row(k, _):
            ik = jnp.full((L,), k, dtype=jnp.int32)
            es = plsc.load_gather(ebuf, [i0, ik])
            for r in range(D // L):
                sl = pl.ds(r * L, L)
                rows_v[b, k, sl] = rows_v[b, k, sl] * es
            return 0
        for q in range(NSPLIT):
            lax.fori_loop(q * HB, (q + 1) * HB, _row, 0, unroll=5)
            pltpu.async_copy(rows_v.at[b, pl.ds(q * HB, HB)],
                             shared_u.at[didx.at[NSPLIT * m3 + q]], sems,
                             add=True)
        return 0
    lax.fori_loop(0, NCB, _chunk_b, 0)

    bl = lax.rem(NCB - 1, 2)
    ml = lax.rem(NCB - 1, NMETA)
    for q in range(NSPLIT):
        pltpu.make_async_copy(rows_v.at[bl, pl.ds(q * HB, HB)],
                              shared_u.at[didx.at[NSPLIT * ml + q]],
                              sems).wait()

    plsc.subcore_barrier()

    dbase = sid * DSTRIPE
    pltpu.sync_copy(shared_u.at[pl.ds(dbase, DSTRIPE)],
                    u_out.at[cid, pl.ds(dbase, DSTRIPE)])

    @pl.when(sid == 0)
    def _drain_tail():
        pltpu.sync_copy(shared_u.at[pl.ds(NS * DSTRIPE, DREM)],
                        u_out.at[cid, pl.ds(NS * DSTRIPE, DREM)])


def _sc_scatter(xc, ei, e2):
    mesh = plsc.VectorSubcoreMesh(core_axis_name="c", subcore_axis_name="s")
    fn = pl.kernel(
        _sc_scatter_body,
        out_type=jax.ShapeDtypeStruct((NC, N_JOB, D), jnp.float32),
        mesh=mesh,
        scratch_types=[
            pltpu.VMEM((NMETA, KB), jnp.int32),
            pltpu.VMEM((NSPLIT * NMETA, KB // NSPLIT), jnp.int32),
            pltpu.VMEM((NMETA, KB), jnp.float32),
            pltpu.VMEM((2, KB, D), jnp.float32),
            pltpu.VMEM_SHARED((N_JOB, D), jnp.float32),
            pltpu.SemaphoreType.DMA,
            pltpu.SemaphoreType.DMA,
            pltpu.SemaphoreType.DMA,
        ],
        compiler_params=pltpu.CompilerParams(needs_layout_passes=False),
    )
    return fn(xc, ei, e2)


def _final_body(u_ref, s_ref, w_ref, b_ref, o_ref):
    u = u_ref[0] + u_ref[1]
    r = jnp.maximum(u, 0.0)
    y = jnp.dot(r, w_ref[...], preferred_element_type=jnp.float32)
    s = jnp.sum(s_ref[...], axis=1, keepdims=True)
    o_ref[...] = (y / (s + 1e-16) + b_ref[0, 0])[:, 0]


def _final(u_parts, s_parts_t, mlp_w, mlp_b):
    return pl.pallas_call(
        _final_body,
        out_shape=jax.ShapeDtypeStruct((N_JOB,), jnp.float32),
    )(u_parts, s_parts_t, mlp_w, mlp_b)


@jax.jit
def kernel(x_job, x_company, edge_index_job_to_company, edge_index_company_to_job,
           proj_job_w, proj_job_b, proj_co_w, proj_co_b,
           lin_src_j2c, lin_dst_j2c, lin_src_c2j, lin_dst_c2j,
           k_lin_w, k_lin_b, q, mlp_w, mlp_b):
    del edge_index_job_to_company, lin_src_j2c, lin_dst_j2c, k_lin_w, k_lin_b, q

    ei = edge_index_company_to_job.reshape(2 * E)

    xc, a_src, a_dst = _proj(
        x_company, proj_co_w, proj_co_b.reshape(1, D),
        x_job, proj_job_w, proj_job_b.reshape(1, D),
        lin_src_c2j.reshape(D, 1), lin_dst_c2j.reshape(D, 1))

    e2, s_parts = _sc_logits(a_src, a_dst, ei)
    u_parts = _sc_scatter(xc, ei, e2)

    return _final(u_parts, s_parts.T, mlp_w, mlp_b.reshape(1, 1))
